# baseline (device time: 161408 ns/iter reference)
import jax
import jax.numpy as jnp
from jax import lax
from jax.experimental import pallas as pl
from jax.experimental.pallas import tpu as pltpu

N_DEV = 4


def kernel(x, w_mat, scale_x, scale_w):
    m_total, k_per = x.shape
    _, n = w_mat.shape
    m_per = m_total // N_DEV

    def body(x_ref, w_ref, sx_ref, sw_ref, out_ref,
             comm_ref, send_sems, recv_sems):
        my = lax.axis_index("i")
        left = lax.rem(my + (N_DEV - 1), N_DEV)
        right = lax.rem(my + 1, N_DEV)

        barrier_sem = pltpu.get_barrier_semaphore()
        pl.semaphore_signal(barrier_sem, inc=1, device_id=(left,),
                            device_id_type=pl.DeviceIdType.MESH)
        pl.semaphore_signal(barrier_sem, inc=1, device_id=(right,),
                            device_id_type=pl.DeviceIdType.MESH)
        pl.semaphore_wait(barrier_sem, 2)

        def partial_f32(c):
            xa = x_ref[pl.ds(c * m_per, m_per), :]
            p = lax.dot_general(
                xa, w_ref[:, :],
                dimension_numbers=(((1,), (0,)), ((), ())),
                preferred_element_type=jnp.int32,
            )
            return p.astype(jnp.float32)

        c0 = lax.rem(my + (N_DEV - 1), N_DEV)
        comm_ref[0, :, :] = partial_f32(c0).astype(jnp.bfloat16)

        for h in range(N_DEV - 1):
            send_slot = h % 2
            recv_slot = (h + 1) % 2
            rdma = pltpu.make_async_remote_copy(
                src_ref=comm_ref.at[send_slot],
                dst_ref=comm_ref.at[recv_slot],
                send_sem=send_sems.at[send_slot],
                recv_sem=recv_sems.at[recv_slot],
                device_id=(right,),
                device_id_type=pl.DeviceIdType.MESH,
            )
            rdma.start()
            c_in = lax.rem(my + (N_DEV - 2 - h), N_DEV)
            part = partial_f32(c_in)
            rdma.wait()
            acc = comm_ref[recv_slot, :, :].astype(jnp.float32) + part
            if h < N_DEV - 2:
                comm_ref[recv_slot, :, :] = acc.astype(jnp.bfloat16)
            else:
                y = acc * (sx_ref[0] * sw_ref[0])
                out_ref[:, :] = y * jax.nn.sigmoid(y)

    return pl.pallas_call(
        body,
        out_shape=jax.ShapeDtypeStruct((m_per, n), jnp.float32),
        in_specs=[
            pl.BlockSpec(memory_space=pltpu.VMEM),
            pl.BlockSpec(memory_space=pltpu.VMEM),
            pl.BlockSpec(memory_space=pltpu.SMEM),
            pl.BlockSpec(memory_space=pltpu.SMEM),
        ],
        out_specs=pl.BlockSpec(memory_space=pltpu.VMEM),
        scratch_shapes=[
            pltpu.VMEM((2, m_per, n), jnp.bfloat16),
            pltpu.SemaphoreType.DMA((2,)),
            pltpu.SemaphoreType.DMA((2,)),
        ],
        compiler_params=pltpu.CompilerParams(collective_id=0),
    )(x, w_mat, scale_x, scale_w)


# device time: 114145 ns/iter; 1.4141x vs baseline; 1.4141x over previous
import jax
import jax.numpy as jnp
from jax import lax
from jax.experimental import pallas as pl
from jax.experimental.pallas import tpu as pltpu

N_DEV = 4
N_HALF = 2


def kernel(x, w_mat, scale_x, scale_w):
    m_total, k_per = x.shape
    _, n = w_mat.shape
    m_per = m_total // N_DEV
    m_half = m_per // N_HALF

    def body(x_ref, w_ref, sx_ref, sw_ref, out_ref,
             send_buf, recv_buf, send_sems, recv_sems):
        my = lax.axis_index("i")
        right = lax.rem(my + 1, N_DEV)
        diag = lax.rem(my + 2, N_DEV)
        left = lax.rem(my + 3, N_DEV)

        barrier_sem = pltpu.get_barrier_semaphore()
        for nbr in (left, right, diag):
            pl.semaphore_signal(barrier_sem, inc=1, device_id=(nbr,),
                                device_id_type=pl.DeviceIdType.MESH)
        pl.semaphore_wait(barrier_sem, 3)

        def half_partial(c, h, out_dtype):
            xa = x_ref[pl.ds(c * m_per + h * m_half, m_half), :]
            p = lax.dot_general(
                xa, w_ref[:, :],
                dimension_numbers=(((1,), (0,)), ((), ())),
                preferred_element_type=jnp.int32,
            )
            return p.astype(out_dtype)

        dests = [(right, 0), (left, 1), (diag, 2)]

        started = []

        def issue(i, h):
            tgt, slot = dests[i]
            send_buf[i, h, :, :] = half_partial(tgt, h, jnp.bfloat16)
            rdma = pltpu.make_async_remote_copy(
                src_ref=send_buf.at[i, h],
                dst_ref=recv_buf.at[slot, h],
                send_sem=send_sems.at[i, h],
                recv_sem=recv_sems.at[slot, h],
                device_id=(tgt,),
                device_id_type=pl.DeviceIdType.MESH,
            )
            rdma.start()
            started.append(rdma)

        for h in range(N_HALF):
            for i in range(3):
                issue(i, h)

        scale = sx_ref[0] * sw_ref[0]
        for h in range(N_HALF):
            out_ref[pl.ds(h * m_half, m_half), :] = half_partial(
                my, h, jnp.float32)

        for h in range(N_HALF):
            for slot in range(3):
                pltpu.make_async_remote_copy(
                    src_ref=send_buf.at[0, 0],
                    dst_ref=recv_buf.at[slot, h],
                    send_sem=send_sems.at[0, 0],
                    recv_sem=recv_sems.at[slot, h],
                    device_id=(right,),
                    device_id_type=pl.DeviceIdType.MESH,
                ).wait_recv()
            acc = (out_ref[pl.ds(h * m_half, m_half), :]
                   + recv_buf[0, h, :, :].astype(jnp.float32)
                   + recv_buf[1, h, :, :].astype(jnp.float32)
                   + recv_buf[2, h, :, :].astype(jnp.float32))
            y = acc * scale
            out_ref[pl.ds(h * m_half, m_half), :] = y * jax.nn.sigmoid(y)

        for rdma in started:
            rdma.wait_send()

    return pl.pallas_call(
        body,
        out_shape=jax.ShapeDtypeStruct((m_per, n), jnp.float32),
        in_specs=[
            pl.BlockSpec(memory_space=pltpu.VMEM),
            pl.BlockSpec(memory_space=pltpu.VMEM),
            pl.BlockSpec(memory_space=pltpu.SMEM),
            pl.BlockSpec(memory_space=pltpu.SMEM),
        ],
        out_specs=pl.BlockSpec(memory_space=pltpu.VMEM),
        scratch_shapes=[
            pltpu.VMEM((3, N_HALF, m_half, n), jnp.bfloat16),
            pltpu.VMEM((3, N_HALF, m_half, n), jnp.bfloat16),
            pltpu.SemaphoreType.DMA((3, N_HALF)),
            pltpu.SemaphoreType.DMA((3, N_HALF)),
        ],
        compiler_params=pltpu.CompilerParams(
            collective_id=0,
            vmem_limit_bytes=100 * 1024 * 1024,
        ),
    )(x, w_mat, scale_x, scale_w)


# device time: 111186 ns/iter; 1.4517x vs baseline; 1.0266x over previous
import jax
import jax.numpy as jnp
from jax import lax
from jax.experimental import pallas as pl
from jax.experimental.pallas import tpu as pltpu

N_DEV = 4
N_HALF = 4


def kernel(x, w_mat, scale_x, scale_w):
    m_total, k_per = x.shape
    _, n = w_mat.shape
    m_per = m_total // N_DEV
    m_half = m_per // N_HALF

    def body(x_ref, w_ref, sx_ref, sw_ref, out_ref,
             send_buf, recv_buf, send_sems, recv_sems):
        my = lax.axis_index("i")
        right = lax.rem(my + 1, N_DEV)
        diag = lax.rem(my + 2, N_DEV)
        left = lax.rem(my + 3, N_DEV)

        barrier_sem = pltpu.get_barrier_semaphore()
        for nbr in (left, right, diag):
            pl.semaphore_signal(barrier_sem, inc=1, device_id=(nbr,),
                                device_id_type=pl.DeviceIdType.MESH)
        pl.semaphore_wait(barrier_sem, 3)

        def half_partial(c, h, out_dtype):
            xa = x_ref[pl.ds(c * m_per + h * m_half, m_half), :]
            p = lax.dot_general(
                xa, w_ref[:, :],
                dimension_numbers=(((1,), (0,)), ((), ())),
                preferred_element_type=jnp.int32,
            )
            return p.astype(out_dtype)

        dests = [(right, 0), (left, 1), (diag, 2)]

        started = []

        def issue(i, h):
            tgt, slot = dests[i]
            send_buf[i, h, :, :] = half_partial(tgt, h, jnp.bfloat16)
            rdma = pltpu.make_async_remote_copy(
                src_ref=send_buf.at[i, h],
                dst_ref=recv_buf.at[slot, h],
                send_sem=send_sems.at[i, h],
                recv_sem=recv_sems.at[slot, h],
                device_id=(tgt,),
                device_id_type=pl.DeviceIdType.MESH,
            )
            rdma.start()
            started.append(rdma)

        for h in range(N_HALF):
            for i in range(3):
                issue(i, h)

        scale = sx_ref[0] * sw_ref[0]
        for h in range(N_HALF):
            out_ref[pl.ds(h * m_half, m_half), :] = half_partial(
                my, h, jnp.float32)

        for h in range(N_HALF):
            for slot in range(3):
                pltpu.make_async_remote_copy(
                    src_ref=send_buf.at[0, 0],
                    dst_ref=recv_buf.at[slot, h],
                    send_sem=send_sems.at[0, 0],
                    recv_sem=recv_sems.at[slot, h],
                    device_id=(right,),
                    device_id_type=pl.DeviceIdType.MESH,
                ).wait_recv()
            acc = (out_ref[pl.ds(h * m_half, m_half), :]
                   + recv_buf[0, h, :, :].astype(jnp.float32)
                   + recv_buf[1, h, :, :].astype(jnp.float32)
                   + recv_buf[2, h, :, :].astype(jnp.float32))
            y = acc * scale
            out_ref[pl.ds(h * m_half, m_half), :] = y * jax.nn.sigmoid(y)

        for rdma in started:
            rdma.wait_send()

    return pl.pallas_call(
        body,
        out_shape=jax.ShapeDtypeStruct((m_per, n), jnp.float32),
        in_specs=[
            pl.BlockSpec(memory_space=pltpu.VMEM),
            pl.BlockSpec(memory_space=pltpu.VMEM),
            pl.BlockSpec(memory_space=pltpu.SMEM),
            pl.BlockSpec(memory_space=pltpu.SMEM),
        ],
        out_specs=pl.BlockSpec(memory_space=pltpu.VMEM),
        scratch_shapes=[
            pltpu.VMEM((3, N_HALF, m_half, n), jnp.bfloat16),
            pltpu.VMEM((3, N_HALF, m_half, n), jnp.bfloat16),
            pltpu.SemaphoreType.DMA((3, N_HALF)),
            pltpu.SemaphoreType.DMA((3, N_HALF)),
        ],
        compiler_params=pltpu.CompilerParams(
            collective_id=0,
            vmem_limit_bytes=100 * 1024 * 1024,
        ),
    )(x, w_mat, scale_x, scale_w)


# device time: 89878 ns/iter; 1.7959x vs baseline; 1.2371x over previous
import jax
import jax.numpy as jnp
from jax import lax
from jax.experimental import pallas as pl
from jax.experimental.pallas import tpu as pltpu

N_DEV = 4
K_SUB = 2


def kernel(x, w_mat, scale_x, scale_w):
    m_total, k_per = x.shape
    _, n = w_mat.shape
    m_per = m_total // N_DEV
    k_sub = k_per // K_SUB

    def body(x_ref, w_ref, sx_ref, sw_ref, out_ref,
             xbuf, wbuf, acc_ref, send_sems, recv_sems):
        my = lax.axis_index("i")
        right = lax.rem(my + 1, N_DEV)
        diag = lax.rem(my + 2, N_DEV)
        left = lax.rem(my + 3, N_DEV)

        barrier_sem = pltpu.get_barrier_semaphore()
        for nbr in (left, right, diag):
            pl.semaphore_signal(barrier_sem, inc=1, device_id=(nbr,),
                                device_id_type=pl.DeviceIdType.MESH)
        pl.semaphore_wait(barrier_sem, 3)

        dests = [(right, 0), (left, 1), (diag, 2)]

        started = []

        def send(src_ref, dst_ref, kind, i, h, tgt):
            rdma = pltpu.make_async_remote_copy(
                src_ref=src_ref,
                dst_ref=dst_ref,
                send_sem=send_sems.at[kind, i, h],
                recv_sem=recv_sems.at[kind, dests[i][1], h],
                device_id=(tgt,),
                device_id_type=pl.DeviceIdType.MESH,
            )
            rdma.start()
            started.append(rdma)

        for h in range(K_SUB):
            for i, (tgt, slot) in enumerate(dests):
                send(
                    x_ref.at[pl.ds(tgt * m_per, m_per),
                             pl.ds(h * k_sub, k_sub)],
                    xbuf.at[slot, h], 0, i, h, tgt,
                )
                send(
                    w_ref.at[pl.ds(h * k_sub, k_sub), :],
                    wbuf.at[slot, h], 1, i, h, tgt,
                )

        def dot_i32(xa, wb):
            return lax.dot_general(
                xa, wb,
                dimension_numbers=(((1,), (0,)), ((), ())),
                preferred_element_type=jnp.int32,
            )

        acc_ref[:, :] = dot_i32(
            x_ref[pl.ds(my * m_per, m_per), :], w_ref[:, :])

        def wait_recv(kind, slot, h):
            pltpu.make_async_remote_copy(
                src_ref=xbuf.at[0, 0],
                dst_ref=xbuf.at[slot, h] if kind == 0 else wbuf.at[slot, h],
                send_sem=send_sems.at[0, 0, 0],
                recv_sem=recv_sems.at[kind, slot, h],
                device_id=(right,),
                device_id_type=pl.DeviceIdType.MESH,
            ).wait_recv()

        for h in range(K_SUB):
            for slot in range(3):
                wait_recv(0, slot, h)
                wait_recv(1, slot, h)
                acc_ref[:, :] = acc_ref[:, :] + dot_i32(
                    xbuf[slot, h], wbuf[slot, h])

        y = acc_ref[:, :].astype(jnp.float32) * (sx_ref[0] * sw_ref[0])
        out_ref[:, :] = y * jax.nn.sigmoid(y)

        for rdma in started:
            rdma.wait_send()

    return pl.pallas_call(
        body,
        out_shape=jax.ShapeDtypeStruct((m_per, n), jnp.float32),
        in_specs=[
            pl.BlockSpec(memory_space=pltpu.VMEM),
            pl.BlockSpec(memory_space=pltpu.VMEM),
            pl.BlockSpec(memory_space=pltpu.SMEM),
            pl.BlockSpec(memory_space=pltpu.SMEM),
        ],
        out_specs=pl.BlockSpec(memory_space=pltpu.VMEM),
        scratch_shapes=[
            pltpu.VMEM((3, K_SUB, m_per, k_sub), jnp.int8),
            pltpu.VMEM((3, K_SUB, k_sub, n), jnp.int8),
            pltpu.VMEM((m_per, n), jnp.int32),
            pltpu.SemaphoreType.DMA((2, 3, K_SUB)),
            pltpu.SemaphoreType.DMA((2, 3, K_SUB)),
        ],
        compiler_params=pltpu.CompilerParams(
            collective_id=0,
            vmem_limit_bytes=100 * 1024 * 1024,
        ),
    )(x, w_mat, scale_x, scale_w)
